# baseline (speedup 1.0000x reference)
import jax
import jax.numpy as jnp
from jax.experimental import pallas as pl
from jax.experimental.pallas import tpu as pltpu

_XS = 2
_WS = 4


def _body(dq_ref, b1_ref, b2_ref, x_hbm, w1_hbm, w2_hbm, emb_hbm,
          zq_ref, ind_ref, diff_ref,
          xv_ref, w1v_ref, w2v_ref, embv_ref, *sems):
    step = pl.program_id(0)
    b = xv_ref.shape[0]
    inp = xv_ref.shape[1]
    dm, ncodes = embv_ref.shape

    copies = []
    xr = b // _XS
    for i in range(_XS):
        copies.append(pltpu.make_async_copy(
            x_hbm.at[pl.ds(i * xr, xr)], xv_ref.at[pl.ds(i * xr, xr)],
            sems[i]))
    wr = inp // _WS
    for i in range(_WS):
        copies.append(pltpu.make_async_copy(
            w1_hbm.at[pl.ds(i * wr, wr)], w1v_ref.at[pl.ds(i * wr, wr)],
            sems[_XS + i]))
    copies.append(pltpu.make_async_copy(w2_hbm, w2v_ref, sems[_XS + _WS]))
    copies.append(pltpu.make_async_copy(emb_hbm, embv_ref,
                                        sems[_XS + _WS + 1]))

    @pl.when(step == 0)
    def _start():
        for c in copies:
            c.start()

    @pl.when(step == 1)
    def _compute():
        for c in copies:
            c.wait()

        h = jnp.dot(xv_ref[...], w1v_ref[...]) + b1_ref[...]
        h = jnp.where(h >= 0, h, 0.01 * h)
        z = jnp.dot(h, w2v_ref[...]) + b2_ref[...]

        emb = embv_ref[...]
        zsq = (z ** 2).sum(axis=1, keepdims=True)
        esq = (emb ** 2).sum(axis=0, keepdims=True)
        dist = zsq - 2.0 * jnp.dot(z, emb) + esq

        minval = jnp.min(dist, axis=1, keepdims=True)
        iota = jax.lax.broadcasted_iota(jnp.int32, (b, ncodes), 1)
        ind = jnp.min(jnp.where(dist == minval, iota, ncodes), axis=1)

        onehot = (iota == ind[:, None]).astype(jnp.float32)
        q = jax.lax.dot_general(onehot, emb, (((1,), (1,)), ((), ())))

        dq = dq_ref[0] != 0
        zq_ref[...] = jnp.where(dq, q, z)
        ind_ref[...] = ind.reshape(1, b)
        diff_ref[0, 0] = jnp.where(dq, jnp.sum(minval) / (b * dm), 0.0)


def _encode(dq, x, w1, b1, b2, w2, emb, *, interpret=False):
    b, inp = x.shape
    dh = w1.shape[1]
    dm, ncodes = emb.shape
    zq, ind, diff = pl.pallas_call(
        _body,
        grid=(2,),
        in_specs=[
            pl.BlockSpec(memory_space=pltpu.SMEM),
            pl.BlockSpec((1, dh), lambda k: (0, 0)),
            pl.BlockSpec((1, dm), lambda k: (0, 0)),
            pl.BlockSpec(memory_space=pl.ANY),
            pl.BlockSpec(memory_space=pl.ANY),
            pl.BlockSpec(memory_space=pl.ANY),
            pl.BlockSpec(memory_space=pl.ANY),
        ],
        out_specs=[
            pl.BlockSpec((b, dm), lambda k: (0, 0)),
            pl.BlockSpec((1, b), lambda k: (0, 0)),
            pl.BlockSpec(memory_space=pltpu.SMEM),
        ],
        out_shape=[
            jax.ShapeDtypeStruct((b, dm), jnp.float32),
            jax.ShapeDtypeStruct((1, b), jnp.int32),
            jax.ShapeDtypeStruct((1, 1), jnp.float32),
        ],
        scratch_shapes=[
            pltpu.VMEM((b, inp), jnp.float32),
            pltpu.VMEM((inp, dh), jnp.float32),
            pltpu.VMEM((dh, dm), jnp.float32),
            pltpu.VMEM((dm, ncodes), jnp.float32),
        ] + [pltpu.SemaphoreType.DMA] * (_XS + _WS + 2),
        compiler_params=pltpu.CompilerParams(
            dimension_semantics=("arbitrary",),
        ),
        interpret=interpret,
    )(dq, b1, b2, x, w1, w2, emb)
    return zq, ind, diff


def kernel(x, W1, b1, W2, b2, embed, do_quantize, k):
    b = x.shape[0]
    xin = x.reshape((b, -1))
    dq = jnp.asarray(do_quantize, jnp.int32).reshape(1)
    zq, ind, diff = _encode(
        dq, xin, W1, b1.reshape(1, -1), b2.reshape(1, -1), W2, embed)
    return zq, diff.reshape(()), ind


# speedup vs baseline: 1.1699x; 1.1699x over previous
import jax
import jax.numpy as jnp
from jax.experimental import pallas as pl
from jax.experimental.pallas import tpu as pltpu


def _body(dq_ref, x_ref, w1_ref, b1_ref, b2_ref, w2_hbm, emb_hbm,
          zq_ref, ind_ref, diff_ref,
          hacc_ref, w2_ref, emb_ref, sem_w2, sem_emb):
    k = pl.program_id(0)
    nk = pl.num_programs(0)
    bm = x_ref.shape[0]

    w2_copy = pltpu.make_async_copy(w2_hbm, w2_ref, sem_w2)
    emb_copy = pltpu.make_async_copy(emb_hbm, emb_ref, sem_emb)

    @pl.when(k == 0)
    def _start():
        w2_copy.start()
        emb_copy.start()

    partial = jnp.dot(x_ref[...], w1_ref[...])

    @pl.when(k == 0)
    def _first():
        hacc_ref[...] = partial

    @pl.when(k > 0)
    def _rest():
        hacc_ref[...] += partial

    @pl.when(k == nk - 1)
    def _tail():
        w2_copy.wait()
        emb_copy.wait()

        h = hacc_ref[...] + b1_ref[...]
        h = jnp.where(h >= 0, h, 0.01 * h)
        z = jnp.dot(h, w2_ref[...]) + b2_ref[...]

        emb = emb_ref[...]
        ncodes = emb.shape[1]
        dm = emb.shape[0]
        zsq = (z ** 2).sum(axis=1, keepdims=True)
        esq = (emb ** 2).sum(axis=0, keepdims=True)
        dist = zsq - 2.0 * jnp.dot(z, emb) + esq

        negd = -dist
        ind = jnp.argmax(negd, axis=1).astype(jnp.int32)
        minval = -jnp.max(negd, axis=1, keepdims=True)
        iota = jax.lax.broadcasted_iota(jnp.int32, (bm, ncodes), 1)

        onehot = (iota == ind[:, None]).astype(jnp.float32)
        q = jax.lax.dot_general(onehot, emb, (((1,), (1,)), ((), ())))

        dq = dq_ref[0] != 0
        zq_ref[...] = jnp.where(dq, q, z)
        ind_ref[...] = ind.reshape(1, bm)
        diff_ref[0, 0] = jnp.where(dq, jnp.sum(minval) / (bm * dm), 0.0)


def _encode(dq, x, w1, b1, b2, w2, emb, *, kc=1024, interpret=False):
    b, inp = x.shape
    dh = w1.shape[1]
    dm, ncodes = emb.shape
    nk = inp // kc
    zq, ind, diff = pl.pallas_call(
        _body,
        grid=(nk,),
        in_specs=[
            pl.BlockSpec(memory_space=pltpu.SMEM),
            pl.BlockSpec((b, kc), lambda k: (0, k)),
            pl.BlockSpec((kc, dh), lambda k: (k, 0)),
            pl.BlockSpec((1, dh), lambda k: (0, 0)),
            pl.BlockSpec((1, dm), lambda k: (0, 0)),
            pl.BlockSpec(memory_space=pl.ANY),
            pl.BlockSpec(memory_space=pl.ANY),
        ],
        out_specs=[
            pl.BlockSpec((b, dm), lambda k: (0, 0)),
            pl.BlockSpec((1, b), lambda k: (0, 0)),
            pl.BlockSpec(memory_space=pltpu.SMEM),
        ],
        out_shape=[
            jax.ShapeDtypeStruct((b, dm), jnp.float32),
            jax.ShapeDtypeStruct((1, b), jnp.int32),
            jax.ShapeDtypeStruct((1, 1), jnp.float32),
        ],
        scratch_shapes=[
            pltpu.VMEM((b, dh), jnp.float32),
            pltpu.VMEM((dh, dm), jnp.float32),
            pltpu.VMEM((dm, ncodes), jnp.float32),
            pltpu.SemaphoreType.DMA,
            pltpu.SemaphoreType.DMA,
        ],
        compiler_params=pltpu.CompilerParams(
            dimension_semantics=("arbitrary",),
        ),
        interpret=interpret,
    )(dq, x, w1, b1, b2, w2, emb)
    return zq, ind, diff


def kernel(x, W1, b1, W2, b2, embed, do_quantize, k):
    b = x.shape[0]
    xin = x.reshape((b, -1))
    dq = jnp.asarray(do_quantize, jnp.int32).reshape(1)
    zq, ind, diff = _encode(
        dq, xin, W1, b1.reshape(1, -1), b2.reshape(1, -1), W2, embed)
    return zq, diff.reshape(()), ind


# speedup vs baseline: 1.1994x; 1.0252x over previous
import jax
import jax.numpy as jnp
from jax.experimental import pallas as pl
from jax.experimental.pallas import tpu as pltpu


def _body(dq_ref, x_ref, w1_ref, b1_ref, b2_ref, w2_hbm, emb_hbm,
          zq_ref, ind_ref, diff_ref,
          hacc_ref, w2_ref, emb_ref, sem_w2, sem_emb):
    k = pl.program_id(0)
    nk = pl.num_programs(0)
    bm = x_ref.shape[0]

    w2_copy = pltpu.make_async_copy(w2_hbm, w2_ref, sem_w2)
    emb_copy = pltpu.make_async_copy(emb_hbm, emb_ref, sem_emb)

    @pl.when(k == 0)
    def _start():
        w2_copy.start()
        emb_copy.start()

    partial = jnp.dot(x_ref[...], w1_ref[...])

    @pl.when(k == 0)
    def _first():
        hacc_ref[...] = partial

    @pl.when((k > 0) & (k < nk - 1))
    def _rest():
        hacc_ref[...] += partial

    @pl.when(k == nk - 1)
    def _tail():
        w2_copy.wait()
        emb_copy.wait()

        h = (hacc_ref[...] + partial) + b1_ref[...]
        h = jnp.where(h >= 0, h, 0.01 * h)
        z = jnp.dot(h, w2_ref[...]) + b2_ref[...]

        emb = emb_ref[...]
        ncodes = emb.shape[1]
        dm = emb.shape[0]
        zsq = (z ** 2).sum(axis=1, keepdims=True)
        esq = (emb ** 2).sum(axis=0, keepdims=True)
        dist = zsq - 2.0 * jnp.dot(z, emb) + esq

        negd = -dist
        ind = jnp.argmax(negd, axis=1).astype(jnp.int32)
        minval = -jnp.max(negd, axis=1, keepdims=True)
        iota = jax.lax.broadcasted_iota(jnp.int32, (bm, ncodes), 1)

        onehot = (iota == ind[:, None]).astype(jnp.float32)
        q = jax.lax.dot_general(onehot, emb, (((1,), (1,)), ((), ())))

        dq = dq_ref[0] != 0
        zq_ref[...] = jnp.where(dq, q, z)
        ind_ref[...] = ind.reshape(1, bm)
        diff_ref[0, 0] = jnp.where(dq, jnp.sum(minval) / (bm * dm), 0.0)


def _encode(dq, x, w1, b1, b2, w2, emb, *, kc=1024, interpret=False):
    b, inp = x.shape
    dh = w1.shape[1]
    dm, ncodes = emb.shape
    nk = inp // kc
    zq, ind, diff = pl.pallas_call(
        _body,
        grid=(nk,),
        in_specs=[
            pl.BlockSpec(memory_space=pltpu.SMEM),
            pl.BlockSpec((b, kc), lambda k: (0, k)),
            pl.BlockSpec((kc, dh), lambda k: (k, 0)),
            pl.BlockSpec((1, dh), lambda k: (0, 0)),
            pl.BlockSpec((1, dm), lambda k: (0, 0)),
            pl.BlockSpec(memory_space=pl.ANY),
            pl.BlockSpec(memory_space=pl.ANY),
        ],
        out_specs=[
            pl.BlockSpec((b, dm), lambda k: (0, 0)),
            pl.BlockSpec((1, b), lambda k: (0, 0)),
            pl.BlockSpec(memory_space=pltpu.SMEM),
        ],
        out_shape=[
            jax.ShapeDtypeStruct((b, dm), jnp.float32),
            jax.ShapeDtypeStruct((1, b), jnp.int32),
            jax.ShapeDtypeStruct((1, 1), jnp.float32),
        ],
        scratch_shapes=[
            pltpu.VMEM((b, dh), jnp.float32),
            pltpu.VMEM((dh, dm), jnp.float32),
            pltpu.VMEM((dm, ncodes), jnp.float32),
            pltpu.SemaphoreType.DMA,
            pltpu.SemaphoreType.DMA,
        ],
        compiler_params=pltpu.CompilerParams(
            dimension_semantics=("arbitrary",),
        ),
        interpret=interpret,
    )(dq, x, w1, b1, b2, w2, emb)
    return zq, ind, diff


def kernel(x, W1, b1, W2, b2, embed, do_quantize, k):
    b = x.shape[0]
    xin = x.reshape((b, -1))
    dq = jnp.asarray(do_quantize, jnp.int32).reshape(1)
    zq, ind, diff = _encode(
        dq, xin, W1, b1.reshape(1, -1), b2.reshape(1, -1), W2, embed)
    return zq, diff.reshape(()), ind
